# Initial kernel scaffold; baseline (speedup 1.0000x reference)
import jax
import jax.numpy as jnp
from jax.experimental import pallas as pl


def kernel(table, x):
    raise NotImplementedError("write your pallas kernel here")


# speedup vs baseline: 5.8298x; 5.8298x over previous
import functools

import jax
import jax.numpy as jnp
from jax import lax
from jax.experimental import pallas as pl
from jax.experimental.pallas import tpu as pltpu
from jax.experimental.pallas import tpu_sc as plsc

VOCAB = 100000
EMB_DIM = 100
BATCH = 16384
SEQ = 50
SEQ_PAD = 56

NUM_WORKERS = 32
B_W = BATCH // NUM_WORKERS
IDX_W = B_W * SEQ_PAD
OUT_W = B_W * EMB_DIM
L = 16
EMB_PAD = 104
NVREG = EMB_DIM // L
TAIL_OFF = EMB_PAD - L


def _count_nonzero(idx_ref, off):
    lane = lax.iota(jnp.int32, 16)
    v0 = idx_ref[pl.ds(off, L)]
    v1 = idx_ref[pl.ds(off + 16, L)]
    v2 = idx_ref[pl.ds(off + 32, L)]
    v3 = idx_ref[pl.ds(off + 34, L)]
    one = jnp.float32(1.0)
    zero = jnp.float32(0.0)
    c = jnp.where(v0 != 0, one, zero)
    c += jnp.where(v1 != 0, one, zero)
    c += jnp.where(v2 != 0, one, zero)
    c += jnp.where((v3 != 0) & (lane >= 14), one, zero)
    for stride in (8, 4, 2, 1):
        c = c + c.at[lane ^ stride].get(mode="promise_in_bounds")
    return c


def _sc_pool(table, x_flat):
    mesh = plsc.VectorSubcoreMesh(core_axis_name="c", subcore_axis_name="s")

    @functools.partial(
        pl.kernel,
        mesh=mesh,
        out_type=jax.ShapeDtypeStruct((BATCH * EMB_DIM,), jnp.float32),
        compiler_params=pltpu.CompilerParams(use_tc_tiling_on_sc=False),
        scratch_types=[
            pltpu.VMEM((IDX_W,), jnp.int32),
            pltpu.VMEM((SEQ, EMB_PAD), jnp.float32),
            pltpu.VMEM((SEQ, EMB_PAD), jnp.float32),
            pltpu.VMEM((OUT_W + L,), jnp.float32),
            pltpu.SemaphoreType.DMA,
            pltpu.SemaphoreType.DMA,
        ],
    )
    def k(table_hbm, x_hbm, out_hbm, idx_v, buf0, buf1, stage_v, sem0, sem1):
        wid = lax.axis_index("s") * 2 + lax.axis_index("c")
        bufs = (buf0, buf1)
        sems = (sem0, sem1)

        pltpu.sync_copy(x_hbm.at[pl.ds(wid * IDX_W, IDX_W)], idx_v)

        def start_gather(b, buf, sem):
            pltpu.make_async_copy(
                table_hbm.at[idx_v.at[pl.ds(b * SEQ_PAD, SEQ)]], buf, sem
            ).start()

        def wait_gather(buf, sem):
            pltpu.make_async_copy(
                table_hbm.at[idx_v.at[pl.ds(0, SEQ)]], buf, sem
            ).wait()

        def process(b, buf):
            cnt = _count_nonzero(idx_v, b * SEQ_PAD)
            scale = 1.0 / (float(SEQ) * jnp.maximum(cnt, 1.0))
            accs = [buf[0, pl.ds(j * L, L)] for j in range(NVREG)]
            acc_t = buf[0, pl.ds(TAIL_OFF, L)]
            for s in range(1, SEQ):
                for j in range(NVREG):
                    accs[j] += buf[s, pl.ds(j * L, L)]
                acc_t += buf[s, pl.ds(TAIL_OFF, L)]
            base = b * EMB_DIM
            for j in range(NVREG):
                stage_v[pl.ds(base + j * L, L)] = accs[j] * scale
            stage_v[pl.ds(base + TAIL_OFF, L)] = acc_t * scale

        start_gather(0, buf0, sem0)
        start_gather(1, buf1, sem1)

        def body(i, carry):
            for par in range(2):
                b = i * 2 + par
                wait_gather(bufs[par], sems[par])
                process(b, bufs[par])
                nb = jnp.minimum(b + 2, B_W - 1)
                start_gather(nb, bufs[par], sems[par])
            return carry

        lax.fori_loop(0, B_W // 2, body, 0)

        wait_gather(buf0, sem0)
        wait_gather(buf1, sem1)

        pltpu.sync_copy(stage_v.at[pl.ds(0, OUT_W)], out_hbm.at[pl.ds(wid * OUT_W, OUT_W)])

    return k(table, x_flat)


def kernel(table, x):
    table = jnp.pad(table.astype(jnp.float32), ((0, 0), (0, EMB_PAD - EMB_DIM)))
    x_pad = jnp.pad(x.astype(jnp.int32), ((0, 0), (0, SEQ_PAD - SEQ)))
    x_flat = x_pad.reshape(-1)
    out = _sc_pool(table, x_flat)
    return out.reshape(BATCH, EMB_DIM)


# speedup vs baseline: 6.4940x; 1.1139x over previous
import functools

import jax
import jax.numpy as jnp
from jax import lax
from jax.experimental import pallas as pl
from jax.experimental.pallas import tpu as pltpu
from jax.experimental.pallas import tpu_sc as plsc

VOCAB = 100000
EMB_DIM = 100
BATCH = 16384
SEQ = 50
SEQ_PAD = 56

NUM_WORKERS = 32
B_W = BATCH // NUM_WORKERS
IDX_W = B_W * SEQ_PAD
OUT_W = B_W * EMB_DIM
L = 16
EMB_PAD = 104
NVREG = EMB_DIM // L
TAIL_OFF = EMB_PAD - L


def _count_nonzero(idx_ref, off):
    lane = lax.iota(jnp.int32, 16)
    v0 = idx_ref[pl.ds(off, L)]
    v1 = idx_ref[pl.ds(off + 16, L)]
    v2 = idx_ref[pl.ds(off + 32, L)]
    v3 = idx_ref[pl.ds(off + 34, L)]
    one = jnp.float32(1.0)
    zero = jnp.float32(0.0)
    c = jnp.where(v0 != 0, one, zero)
    c += jnp.where(v1 != 0, one, zero)
    c += jnp.where(v2 != 0, one, zero)
    c += jnp.where((v3 != 0) & (lane >= 14), one, zero)
    for stride in (8, 4, 2, 1):
        c = c + c.at[lane ^ stride].get(mode="promise_in_bounds")
    return c


def _sc_pool(table, x_flat):
    mesh = plsc.VectorSubcoreMesh(core_axis_name="c", subcore_axis_name="s")

    @functools.partial(
        pl.kernel,
        mesh=mesh,
        out_type=jax.ShapeDtypeStruct((BATCH * EMB_DIM,), jnp.float32),
        compiler_params=pltpu.CompilerParams(use_tc_tiling_on_sc=False),
        scratch_types=[
            pltpu.VMEM((IDX_W,), jnp.int32),
            pltpu.VMEM((SEQ, EMB_PAD), jnp.float32),
            pltpu.VMEM((SEQ, EMB_PAD), jnp.float32),
            pltpu.VMEM((OUT_W + L,), jnp.float32),
            pltpu.SemaphoreType.DMA,
            pltpu.SemaphoreType.DMA,
        ],
    )
    def k(table_hbm, x_hbm, out_hbm, idx_v, buf0, buf1, stage_v, sem0, sem1):
        wid = lax.axis_index("s") * 2 + lax.axis_index("c")
        bufs = (buf0, buf1)
        sems = (sem0, sem1)

        pltpu.sync_copy(x_hbm.at[pl.ds(wid * IDX_W, IDX_W)], idx_v)

        def start_gather(b, buf, sem):
            pltpu.make_async_copy(
                table_hbm.at[idx_v.at[pl.ds(b * SEQ_PAD, SEQ)]], buf, sem
            ).start()

        def wait_gather(buf, sem):
            pltpu.make_async_copy(
                table_hbm.at[idx_v.at[pl.ds(0, SEQ)]], buf, sem
            ).wait()

        def process(b, buf):
            cnt = _count_nonzero(idx_v, b * SEQ_PAD)
            scale = 1.0 / (float(SEQ) * jnp.maximum(cnt, 1.0))
            base = b * EMB_DIM
            for j in list(range(NVREG)) + [None]:
                off = TAIL_OFF if j is None else j * L
                parts = [buf[s0, pl.ds(off, L)] for s0 in range(4)]
                for s in range(4, SEQ):
                    parts[s % 4] += buf[s, pl.ds(off, L)]
                acc = (parts[0] + parts[1]) + (parts[2] + parts[3])
                stage_v[pl.ds(base + off, L)] = acc * scale

        start_gather(0, buf0, sem0)
        start_gather(1, buf1, sem1)

        def body(i, carry):
            for par in range(2):
                b = i * 2 + par
                wait_gather(bufs[par], sems[par])
                process(b, bufs[par])
                nb = jnp.minimum(b + 2, B_W - 1)
                start_gather(nb, bufs[par], sems[par])
            return carry

        lax.fori_loop(0, B_W // 2, body, 0)

        wait_gather(buf0, sem0)
        wait_gather(buf1, sem1)

        pltpu.sync_copy(stage_v.at[pl.ds(0, OUT_W)], out_hbm.at[pl.ds(wid * OUT_W, OUT_W)])

    return k(table, x_flat)


def kernel(table, x):
    table = jnp.pad(table.astype(jnp.float32), ((0, 0), (0, EMB_PAD - EMB_DIM)))
    x_pad = jnp.pad(x.astype(jnp.int32), ((0, 0), (0, SEQ_PAD - SEQ)))
    x_flat = x_pad.reshape(-1)
    out = _sc_pool(table, x_flat)
    return out.reshape(BATCH, EMB_DIM)


# speedup vs baseline: 6.9907x; 1.0765x over previous
import functools

import jax
import jax.numpy as jnp
from jax import lax
from jax.experimental import pallas as pl
from jax.experimental.pallas import tpu as pltpu
from jax.experimental.pallas import tpu_sc as plsc

VOCAB = 100000
EMB_DIM = 100
BATCH = 16384
SEQ = 50
SEQ_PAD = 56

NUM_WORKERS = 32
B_W = BATCH // NUM_WORKERS
IDX_W = B_W * SEQ_PAD
OUT_W = B_W * EMB_DIM
L = 16
EMB_PAD = 128
NVREG = EMB_DIM // L
TAIL_OFF = EMB_DIM - L


def _count_nonzero(idx_ref, off):
    lane = lax.iota(jnp.int32, 16)
    v0 = idx_ref[pl.ds(off, L)]
    v1 = idx_ref[pl.ds(off + 16, L)]
    v2 = idx_ref[pl.ds(off + 32, L)]
    v3 = idx_ref[pl.ds(off + 34, L)]
    one = jnp.float32(1.0)
    zero = jnp.float32(0.0)
    c = jnp.where(v0 != 0, one, zero)
    c += jnp.where(v1 != 0, one, zero)
    c += jnp.where(v2 != 0, one, zero)
    c += jnp.where((v3 != 0) & (lane >= 14), one, zero)
    for stride in (8, 4, 2, 1):
        c = c + c.at[lane ^ stride].get(mode="promise_in_bounds")
    return c


def _sc_pool(table, x_flat):
    mesh = plsc.VectorSubcoreMesh(core_axis_name="c", subcore_axis_name="s")

    @functools.partial(
        pl.kernel,
        mesh=mesh,
        out_type=jax.ShapeDtypeStruct((BATCH * EMB_DIM,), jnp.float32),
        scratch_types=[
            pltpu.VMEM((IDX_W,), jnp.int32),
            pltpu.VMEM((SEQ, EMB_PAD), jnp.float32),
            pltpu.VMEM((SEQ, EMB_PAD), jnp.float32),
            pltpu.VMEM((OUT_W + L,), jnp.float32),
            pltpu.SemaphoreType.DMA,
            pltpu.SemaphoreType.DMA,
        ],
    )
    def k(table_hbm, x_hbm, out_hbm, idx_v, buf0, buf1, stage_v, sem0, sem1):
        wid = lax.axis_index("s") * 2 + lax.axis_index("c")
        bufs = (buf0, buf1)
        sems = (sem0, sem1)

        pltpu.sync_copy(x_hbm.at[pl.ds(wid * IDX_W, IDX_W)], idx_v)

        def start_gather(b, buf, sem):
            pltpu.make_async_copy(
                table_hbm.at[idx_v.at[pl.ds(b * SEQ_PAD, SEQ)]], buf, sem
            ).start()

        def wait_gather(buf, sem):
            pltpu.make_async_copy(
                table_hbm.at[idx_v.at[pl.ds(0, SEQ)]], buf, sem
            ).wait()

        def process(b, buf):
            cnt = _count_nonzero(idx_v, b * SEQ_PAD)
            scale = 1.0 / (float(SEQ) * jnp.maximum(cnt, 1.0))
            base = b * EMB_DIM
            for j in list(range(NVREG)) + [None]:
                off = TAIL_OFF if j is None else j * L
                parts = [buf[s0, pl.ds(off, L)] for s0 in range(4)]
                for s in range(4, SEQ):
                    parts[s % 4] += buf[s, pl.ds(off, L)]
                acc = (parts[0] + parts[1]) + (parts[2] + parts[3])
                stage_v[pl.ds(base + off, L)] = acc * scale

        start_gather(0, buf0, sem0)
        start_gather(1, buf1, sem1)

        def body(i, carry):
            for par in range(2):
                b = i * 2 + par
                wait_gather(bufs[par], sems[par])
                process(b, bufs[par])
                nb = jnp.minimum(b + 2, B_W - 1)
                start_gather(nb, bufs[par], sems[par])
            return carry

        lax.fori_loop(0, B_W // 2, body, 0)

        wait_gather(buf0, sem0)
        wait_gather(buf1, sem1)

        pltpu.sync_copy(stage_v.at[pl.ds(0, OUT_W)], out_hbm.at[pl.ds(wid * OUT_W, OUT_W)])

    return k(table, x_flat)


def kernel(table, x):
    table = jnp.pad(table.astype(jnp.float32), ((0, 0), (0, EMB_PAD - EMB_DIM)))
    x_pad = jnp.pad(x.astype(jnp.int32), ((0, 0), (0, SEQ_PAD - SEQ)))
    x_flat = x_pad.reshape(-1)
    out = _sc_pool(table, x_flat)
    return out.reshape(BATCH, EMB_DIM)


# speedup vs baseline: 8.3011x; 1.1874x over previous
import functools

import jax
import jax.numpy as jnp
from jax import lax
from jax.experimental import pallas as pl
from jax.experimental.pallas import tpu as pltpu
from jax.experimental.pallas import tpu_sc as plsc

VOCAB = 100000
EMB_DIM = 100
BATCH = 16384
SEQ = 50
SEQ_PAD = 56

NUM_WORKERS = 32
B_W = BATCH // NUM_WORKERS
IDX_W = B_W * SEQ_PAD
OUT_W = B_W * EMB_DIM
L = 16
EMB_PAD = 128
NVREG = EMB_DIM // L
TAIL_OFF = EMB_DIM - L


def _count_nonzero(idx_ref, off):
    lane = lax.iota(jnp.int32, 16)
    v0 = idx_ref[pl.ds(off, L)]
    v1 = idx_ref[pl.ds(off + 16, L)]
    v2 = idx_ref[pl.ds(off + 32, L)]
    v3 = idx_ref[pl.ds(off + 34, L)]
    one = jnp.float32(1.0)
    zero = jnp.float32(0.0)
    c = jnp.where(v0 != 0, one, zero)
    c += jnp.where(v1 != 0, one, zero)
    c += jnp.where(v2 != 0, one, zero)
    c += jnp.where((v3 != 0) & (lane >= 14), one, zero)
    for stride in (8, 4, 2, 1):
        c = c + c.at[lane ^ stride].get(mode="promise_in_bounds")
    return c


def _sc_pool(table, x_flat):
    mesh = plsc.VectorSubcoreMesh(core_axis_name="c", subcore_axis_name="s")

    @functools.partial(
        pl.kernel,
        mesh=mesh,
        out_type=jax.ShapeDtypeStruct((BATCH * EMB_DIM,), jnp.float32),
        scratch_types=[
            pltpu.VMEM((IDX_W,), jnp.int32),
            pltpu.VMEM((SEQ, EMB_PAD), jnp.float32),
            pltpu.VMEM((SEQ, EMB_PAD), jnp.float32),
            pltpu.VMEM((SEQ, EMB_PAD), jnp.float32),
            pltpu.VMEM((OUT_W + L,), jnp.float32),
            pltpu.SemaphoreType.DMA,
            pltpu.SemaphoreType.DMA,
            pltpu.SemaphoreType.DMA,
        ],
    )
    def k(table_hbm, x_hbm, out_hbm, idx_v, buf0, buf1, buf2, stage_v,
          sem0, sem1, sem2):
        wid = lax.axis_index("s") * 2 + lax.axis_index("c")
        bufs = (buf0, buf1, buf2)
        sems = (sem0, sem1, sem2)

        pltpu.sync_copy(x_hbm.at[pl.ds(wid * IDX_W, IDX_W)], idx_v)

        def start_gather(b, buf, sem):
            pltpu.make_async_copy(
                table_hbm.at[idx_v.at[pl.ds(b * SEQ_PAD, SEQ)]], buf, sem
            ).start()

        def wait_gather(buf, sem):
            pltpu.make_async_copy(
                table_hbm.at[idx_v.at[pl.ds(0, SEQ)]], buf, sem
            ).wait()

        def process(b, buf):
            cnt = _count_nonzero(idx_v, b * SEQ_PAD)
            scale = 1.0 / (float(SEQ) * jnp.maximum(cnt, 1.0))
            base = b * EMB_DIM
            for j in list(range(NVREG)) + [None]:
                off = TAIL_OFF if j is None else j * L
                parts = [buf[s0, pl.ds(off, L)] for s0 in range(4)]
                for s in range(4, SEQ):
                    parts[s % 4] += buf[s, pl.ds(off, L)]
                acc = (parts[0] + parts[1]) + (parts[2] + parts[3])
                stage_v[pl.ds(base + off, L)] = acc * scale

        for p in range(3):
            start_gather(p, bufs[p], sems[p])

        def body(i, carry):
            for par in range(3):
                b = i * 3 + par
                wait_gather(bufs[par], sems[par])
                process(b, bufs[par])
                nb = jnp.minimum(b + 3, B_W - 1)
                start_gather(nb, bufs[par], sems[par])
            return carry

        lax.fori_loop(0, B_W // 3, body, 0)
        for p in range(3):
            wait_gather(bufs[p], sems[p])
            if p < 2:
                process(B_W - 2 + p, bufs[p])

        pltpu.sync_copy(stage_v.at[pl.ds(0, OUT_W)], out_hbm.at[pl.ds(wid * OUT_W, OUT_W)])

    return k(table, x_flat)


def kernel(table, x):
    table = lax.concatenate(
        [table.astype(jnp.float32),
         jnp.zeros((VOCAB, EMB_PAD - EMB_DIM), jnp.float32)], 1)
    x_pad = jnp.pad(x.astype(jnp.int32), ((0, 0), (0, SEQ_PAD - SEQ)))
    x_flat = x_pad.reshape(-1)
    out = _sc_pool(table, x_flat)
    return out.reshape(BATCH, EMB_DIM)
